# Initial kernel scaffold; baseline (speedup 1.0000x reference)
import jax
import jax.numpy as jnp
from jax.experimental import pallas as pl


def kernel(gfusion1, gfusion2, W_eg, b_eg, W_coo, b_coo, W_pred, b_pred, W_pd, b_pd, W_pr, b_pr, W_out, b_out, bn_gamma, bn_beta):
    raise NotImplementedError("write your pallas kernel here")


# speedup vs baseline: 3.3898x; 3.3898x over previous
import jax
import jax.numpy as jnp
from jax.experimental import pallas as pl

_MID = 64
_KSQ = 9


def _stage1_kernel(g1_ref, g2_ref, weg_ref, beg_ref, wcoo_ref, bcoo_ref,
                   wpred_ref, bpred_ref, wpd_ref, bpd_ref, wpr_ref, bpr_ref,
                   ep_ref, knl_ref, base_ref, ssum_ref):
    t = pl.program_id(1)
    x1 = g1_ref[0]
    x2 = g2_ref[0]
    gate = jax.nn.sigmoid(
        jnp.dot(weg_ref[...], x1, preferred_element_type=jnp.float32)
        + beg_ref[...])
    ef = x1 * gate
    ep = (jnp.dot(wcoo_ref[...], ef, preferred_element_type=jnp.float32)
          + bcoo_ref[...])
    ep_ref[0] = ep
    raw = (jnp.dot(wpred_ref[...], ep, preferred_element_type=jnp.float32)
           + bpred_ref[...])
    score = raw[:_MID]

    @pl.when(t == 0)
    def _():
        ssum_ref[...] = jnp.zeros_like(ssum_ref)

    ssum_ref[...] += jnp.sum(score, axis=1)[None, None, :]

    taps = [raw[_MID + i * _MID:_MID + (i + 1) * _MID] for i in range(_KSQ)]
    m = taps[0]
    for i in range(1, _KSQ):
        m = jnp.maximum(m, taps[i])
    es = [jnp.exp(tp - m) for tp in taps]
    denom = es[0]
    for i in range(1, _KSQ):
        denom = denom + es[i]
    inv = 1.0 / denom
    for i in range(_KSQ):
        knl_ref[0, i] = es[i] * inv

    fd = (jnp.dot(wpd_ref[...], x1, preferred_element_type=jnp.float32)
          + bpd_ref[...])
    fr = (jnp.dot(wpr_ref[...], x2, preferred_element_type=jnp.float32)
          + bpr_ref[...])
    base_ref[0] = jax.nn.relu(fd * fr)


def _stage2_kernel(base_ref, knl_ref, out_ref):
    x = base_ref[0]
    H = x.shape[1]
    W = x.shape[2]
    for _ in range(2):
        xp = jnp.concatenate([x[:, 1:2, :], x, x[:, H - 2:H - 1, :]], axis=1)
        xp = jnp.concatenate([xp[:, :, 1:2], xp, xp[:, :, W - 2:W - 1]],
                             axis=2)
        acc = None
        for i in range(3):
            for j in range(3):
                contrib = xp[:, i:i + H, j:j + W] * knl_ref[0, i * 3 + j]
                acc = contrib if acc is None else acc + contrib
        x = acc
    out_ref[0] = x


def _stage3_kernel(x_ref, mask_ref, wout_ref, bout_ref, gamma_ref, beta_ref,
                   out_ref):
    x = x_ref[0] * mask_ref[0]
    y = (jnp.dot(wout_ref[...], x, preferred_element_type=jnp.float32)
         + bout_ref[...])
    y = y * gamma_ref[...] + beta_ref[...]
    out_ref[0] = jax.nn.relu(y)


def kernel(gfusion1, gfusion2, W_eg, b_eg, W_coo, b_coo, W_pred, b_pred,
           W_pd, b_pd, W_pr, b_pr, W_out, b_out, bn_gamma, bn_beta):
    B, Cin, H, Wd = gfusion1.shape
    out_c = gfusion2.shape[1]
    edge_c = W_coo.shape[0]
    mid, ksq = _MID, _KSQ
    HW = H * Wd
    T = 3584
    nT = HW // T

    g1f = gfusion1.reshape(B, Cin, HW)
    g2f = gfusion2.reshape(B, out_c, HW)

    Wk = W_pred[mid:].reshape(mid, ksq, edge_c).transpose(1, 0, 2)
    Wp = jnp.concatenate([W_pred[:mid], Wk.reshape(mid * ksq, edge_c)], axis=0)
    bk = b_pred[mid:].reshape(mid, ksq).T.reshape(-1)
    bp = jnp.concatenate([b_pred[:mid], bk], axis=0)[:, None]

    full = lambda arr: pl.BlockSpec(arr.shape, lambda b, t: (0,) * arr.ndim)
    weg, beg = W_eg, b_eg[:, None]
    wcoo, bcoo = W_coo, b_coo[:, None]
    wpd, bpd = W_pd, b_pd[:, None]
    wpr, bpr = W_pr, b_pr[:, None]

    ep, knl, base, ssum = pl.pallas_call(
        _stage1_kernel,
        grid=(B, nT),
        in_specs=[
            pl.BlockSpec((1, Cin, T), lambda b, t: (b, 0, t)),
            pl.BlockSpec((1, out_c, T), lambda b, t: (b, 0, t)),
            full(weg), full(beg), full(wcoo), full(bcoo),
            full(Wp), full(bp), full(wpd), full(bpd), full(wpr), full(bpr),
        ],
        out_specs=[
            pl.BlockSpec((1, edge_c, T), lambda b, t: (b, 0, t)),
            pl.BlockSpec((1, ksq, mid, T), lambda b, t: (b, 0, 0, t)),
            pl.BlockSpec((1, mid, T), lambda b, t: (b, 0, t)),
            pl.BlockSpec((1, 8, mid), lambda b, t: (b, 0, 0)),
        ],
        out_shape=[
            jax.ShapeDtypeStruct((B, edge_c, HW), jnp.float32),
            jax.ShapeDtypeStruct((B, ksq, mid, HW), jnp.float32),
            jax.ShapeDtypeStruct((B, mid, HW), jnp.float32),
            jax.ShapeDtypeStruct((B, 8, mid), jnp.float32),
        ],
    )(g1f, g2f, weg, beg, wcoo, bcoo, Wp, bp, wpd, bpd, wpr, bpr)

    scores = jax.nn.sigmoid(ssum[:, 0, :] / HW)
    scale = jnp.clip(jnp.mean(scores), 0.25, 1.0)
    k_t = jnp.maximum(jnp.ceil(mid * scale).astype(jnp.int32), max(4, mid // 4))
    _, idx = jax.lax.top_k(scores, mid)
    keep = jnp.zeros((B, mid), jnp.float32).at[
        jnp.arange(B)[:, None], idx].set(
        (jnp.arange(mid)[None, :] < k_t).astype(jnp.float32))

    base4 = base.reshape(B, mid, H, Wd)
    knl5 = knl.reshape(B, ksq, mid, H, Wd)
    Cb = 4
    xdiff = pl.pallas_call(
        _stage2_kernel,
        grid=(B, mid // Cb),
        in_specs=[
            pl.BlockSpec((1, Cb, H, Wd), lambda b, c: (b, c, 0, 0)),
            pl.BlockSpec((1, ksq, Cb, H, Wd), lambda b, c: (b, 0, c, 0, 0)),
        ],
        out_specs=pl.BlockSpec((1, Cb, H, Wd), lambda b, c: (b, c, 0, 0)),
        out_shape=jax.ShapeDtypeStruct((B, mid, H, Wd), jnp.float32),
    )(base4, knl5)

    gamma = (bn_gamma / jnp.sqrt(1.0 + 1e-5))[:, None]
    out = pl.pallas_call(
        _stage3_kernel,
        grid=(B, nT),
        in_specs=[
            pl.BlockSpec((1, mid, T), lambda b, t: (b, 0, t)),
            pl.BlockSpec((1, mid, 1), lambda b, t: (b, 0, 0)),
            full(W_out), full(b_out[:, None]), full(gamma),
            full(bn_beta[:, None]),
        ],
        out_specs=pl.BlockSpec((1, out_c, T), lambda b, t: (b, 0, t)),
        out_shape=jax.ShapeDtypeStruct((B, out_c, HW), jnp.float32),
    )(xdiff.reshape(B, mid, HW), keep[:, :, None], W_out, b_out[:, None],
      gamma, bn_beta[:, None])

    return out.reshape(B, out_c, H, Wd), ep.reshape(B, edge_c, H, Wd)
